# baseline (speedup 1.0000x reference)
import functools
import math

import jax
import jax.numpy as jnp
from jax import lax
from jax.experimental import pallas as pl
from jax.experimental.pallas import tpu as pltpu
from jax.experimental.pallas import tpu_sc as plsc

N = 10000
E = 320000
D_IN = 128
D_EMB = 256
D_HALF = 128
N_GR = 100
N_GR_PAD = 104
CUTOFF = 6.0
PI2 = math.pi * math.pi

NCHUNK = 32
CHUNK = E // NCHUNK
BIN = 80
NBIN = CHUNK // BIN
BROW = 128
NBC = -(-CHUNK // BROW)
RG = 10
RB = N // RG
NSUB = 16
ROWS_PER_TILE = N // NSUB

_COS_SQRT_COEFS = tuple((-1.0) ** k / math.factorial(2 * k) for k in range(12))


def _cos_sqrt(y):
    acc = jnp.full_like(y, _COS_SQRT_COEFS[-1])
    for c in _COS_SQRT_COEFS[-2::-1]:
        acc = acc * y + c
    return acc


def _dot(a, b):
    return jnp.dot(a, b, preferred_element_type=jnp.float32)


def _tc_embed_body(x_ref, wi_ref, wm_ref, h_ref, m0_ref, m1_ref):
    hb = jax.nn.gelu(_dot(x_ref[...], wi_ref[...]))
    h_ref[...] = hb
    mb = jax.nn.gelu(_dot(hb, wm_ref[...]))
    m0_ref[...] = mb[:, :D_HALF]
    m1_ref[...] = mb[:, D_HALF:]


def _tc_embed(x, W_in, Wm0):
    return pl.pallas_call(
        _tc_embed_body,
        grid=(RG,),
        in_specs=[
            pl.BlockSpec((RB, D_IN), lambda i: (i, 0)),
            pl.BlockSpec((D_IN, D_EMB), lambda i: (0, 0)),
            pl.BlockSpec((D_EMB, D_EMB), lambda i: (0, 0)),
        ],
        out_specs=[
            pl.BlockSpec((RB, D_EMB), lambda i: (i, 0)),
            pl.BlockSpec((RB, D_HALF), lambda i: (i, 0)),
            pl.BlockSpec((RB, D_HALF), lambda i: (i, 0)),
        ],
        out_shape=[
            jax.ShapeDtypeStruct((N, D_EMB), jnp.float32),
            jax.ShapeDtypeStruct((N, D_HALF), jnp.float32),
            jax.ShapeDtypeStruct((N, D_HALF), jnp.float32),
        ],
    )(x, W_in, Wm0)


def _tc_update_body(h_ref, a0_ref, a1_ref, wut_ref, wub_ref, wm_ref,
                    hn_ref, m0_ref, m1_ref):
    a = _dot(a0_ref[...], wut_ref[...]) + _dot(a1_ref[...], wub_ref[...])
    hn = h_ref[...] + jax.nn.gelu(a)
    hn_ref[...] = hn
    mb = jax.nn.gelu(_dot(hn, wm_ref[...]))
    m0_ref[...] = mb[:, :D_HALF]
    m1_ref[...] = mb[:, D_HALF:]


def _tc_update(h, agg0, agg1, Wu_t, Wu_b, Wm):
    return pl.pallas_call(
        _tc_update_body,
        grid=(RG,),
        in_specs=[
            pl.BlockSpec((RB, D_EMB), lambda i: (i, 0)),
            pl.BlockSpec((RB, D_HALF), lambda i: (i, 0)),
            pl.BlockSpec((RB, D_HALF), lambda i: (i, 0)),
            pl.BlockSpec((D_HALF, D_EMB), lambda i: (0, 0)),
            pl.BlockSpec((D_HALF, D_EMB), lambda i: (0, 0)),
            pl.BlockSpec((D_EMB, D_EMB), lambda i: (0, 0)),
        ],
        out_specs=[
            pl.BlockSpec((RB, D_EMB), lambda i: (i, 0)),
            pl.BlockSpec((RB, D_HALF), lambda i: (i, 0)),
            pl.BlockSpec((RB, D_HALF), lambda i: (i, 0)),
        ],
        out_shape=[
            jax.ShapeDtypeStruct((N, D_EMB), jnp.float32),
            jax.ShapeDtypeStruct((N, D_HALF), jnp.float32),
            jax.ShapeDtypeStruct((N, D_HALF), jnp.float32),
        ],
    )(h, agg0, agg1, Wu_t, Wu_b, Wm)


def _tc_final_body(h_ref, a0_ref, a1_ref, wut_ref, wub_ref, b_ref,
                   w1_ref, w2_ref, w3_ref, out_ref, g_acc):
    i = pl.program_id(0)
    a = _dot(a0_ref[...], wut_ref[...]) + _dot(a1_ref[...], wub_ref[...])
    hn = h_ref[...] + jax.nn.gelu(a)
    seg = b_ref[0]
    onehot = (lax.broadcasted_iota(jnp.int32, (N_GR_PAD, RB), 0) == seg
              ).astype(jnp.float32)

    @pl.when(i == 0)
    def _():
        g_acc[...] = jnp.zeros_like(g_acc)

    g_acc[...] += _dot(onehot, hn)

    @pl.when(i == RG - 1)
    def _():
        g = jax.nn.gelu(_dot(g_acc[...], w1_ref[...]))
        g = jax.nn.gelu(_dot(g, w2_ref[...]))
        out_ref[...] = _dot(g, w3_ref[...])[:N_GR]


def _tc_final(h, agg0, agg1, Wu_t, Wu_b, batch3, W1, W2, W3):
    return pl.pallas_call(
        _tc_final_body,
        grid=(RG,),
        in_specs=[
            pl.BlockSpec((RB, D_EMB), lambda i: (i, 0)),
            pl.BlockSpec((RB, D_HALF), lambda i: (i, 0)),
            pl.BlockSpec((RB, D_HALF), lambda i: (i, 0)),
            pl.BlockSpec((D_HALF, D_EMB), lambda i: (0, 0)),
            pl.BlockSpec((D_HALF, D_EMB), lambda i: (0, 0)),
            pl.BlockSpec((1, 1, RB), lambda i: (i, 0, 0)),
            pl.BlockSpec((D_EMB, D_EMB), lambda i: (0, 0)),
            pl.BlockSpec((D_EMB, D_EMB), lambda i: (0, 0)),
            pl.BlockSpec((D_EMB, D_IN), lambda i: (0, 0)),
        ],
        out_specs=pl.BlockSpec((N_GR, D_IN), lambda i: (0, 0)),
        out_shape=jax.ShapeDtypeStruct((N_GR, D_IN), jnp.float32),
        scratch_shapes=[pltpu.VMEM((N_GR_PAD, D_EMB), jnp.float32)],
    )(h, agg0, agg1, Wu_t, Wu_b, batch3, W1, W2, W3)


_MESH = plsc.VectorSubcoreMesh(core_axis_name="c", subcore_axis_name="s")
_SC_PARAMS = pltpu.CompilerParams(needs_layout_passes=False)


@functools.partial(
    pl.kernel,
    out_type=[
        jax.ShapeDtypeStruct((NCHUNK, NBC, 2 * BROW), jnp.int32),
        jax.ShapeDtypeStruct((NCHUNK, NBC, BROW), jnp.int32),
        jax.ShapeDtypeStruct((NCHUNK, 16), jnp.int32),
    ],
    mesh=_MESH,
    compiler_params=_SC_PARAMS,
    scratch_types=[
        pltpu.VMEM((3 * N,), jnp.float32),
        pltpu.VMEM((NBIN, BIN), jnp.int32),
        pltpu.VMEM((NBIN, BIN), jnp.int32),
        pltpu.VMEM((NBC, 2 * BROW), jnp.int32),
        pltpu.VMEM((NBC, BROW), jnp.int32),
        pltpu.VMEM((16,), jnp.int32),
    ],
)
def _sc_env(pos_hbm, srcR_hbm, dstR_hbm, se_hbm, do_hbm, cnt_hbm,
            pos_v, src_v, dst_v, se_v, do_v, cnt_v):
    c = lax.axis_index("c")
    s = lax.axis_index("s")
    wid = s * 2 + c
    pltpu.sync_copy(pos_hbm, pos_v)
    pltpu.sync_copy(srcR_hbm.at[wid], src_v)
    pltpu.sync_copy(dstR_hbm.at[wid], dst_v)
    iz = jnp.zeros((16,), jnp.int32)
    i1 = jnp.full((16,), 1, jnp.int32)
    i2 = jnp.full((16,), 2, jnp.int32)
    scale = PI2 / (CUTOFF * CUTOFF)

    def zrow(j, carry):
        for k in range(2 * BROW // 16):
            se_v[j, pl.ds(16 * k, 16)] = iz
        for k in range(BROW // 16):
            do_v[j, pl.ds(16 * k, 16)] = iz
        return carry

    lax.fori_loop(0, NBC, zrow, 0)

    def row(j, off):
        for k in range(BIN // 16):
            sl = pl.ds(16 * k, 16)
            sv = src_v[j, sl]
            tv = dst_v[j, sl]
            s3 = sv * 3
            t3 = tv * 3
            dx = (plsc.load_gather(pos_v, [s3 + iz])
                  - plsc.load_gather(pos_v, [t3 + iz]))
            dy = (plsc.load_gather(pos_v, [s3 + i1])
                  - plsc.load_gather(pos_v, [t3 + i1]))
            dz = (plsc.load_gather(pos_v, [s3 + i2])
                  - plsc.load_gather(pos_v, [t3 + i2]))
            s2 = dx * dx + dy * dy + dz * dz + 1e-8
            y = s2 * scale
            ev = 0.5 * (_cos_sqrt(y) + 1.0)
            ev = jnp.maximum(ev, 0.0)
            keep = y < PI2
            kin = jnp.where(keep, 1, 0)
            pos = off + plsc.cumsum(kin) - 1
            jb = pos // BROW
            lane = pos - jb * BROW
            plsc.store_scatter(se_v, [jb, lane], sv, mask=keep)
            plsc.store_scatter(se_v, [jb, lane + BROW],
                               plsc.bitcast(ev, jnp.int32), mask=keep)
            plsc.store_scatter(do_v, [jb, lane], tv, mask=keep)
            off = off + plsc.all_reduce_population_count(keep)
        return off

    off = lax.fori_loop(0, NBIN, row, iz)
    cnt_v[...] = off
    pltpu.sync_copy(se_v, se_hbm.at[wid])
    pltpu.sync_copy(do_v, do_hbm.at[wid])
    pltpu.sync_copy(cnt_v, cnt_hbm.at[wid])


@functools.partial(
    pl.kernel,
    out_type=[
        jax.ShapeDtypeStruct((N, D_HALF), jnp.float32),
        jax.ShapeDtypeStruct((N, D_HALF), jnp.float32),
    ],
    mesh=_MESH,
    compiler_params=_SC_PARAMS,
    scratch_types=[
        pltpu.VMEM((4, 2 * BROW), jnp.int32),
        pltpu.VMEM((4, BROW), jnp.int32),
        pltpu.VMEM((BROW, D_HALF), jnp.float32),
        pltpu.VMEM((BROW, D_HALF), jnp.float32),
        pltpu.VMEM((125, D_HALF), jnp.float32),
        pltpu.VMEM((16,), jnp.int32),
        pltpu.VMEM_SHARED((N, D_HALF), jnp.float32),
        pltpu.SemaphoreType.DMA,
        pltpu.SemaphoreType.DMA,
        pltpu.SemaphoreType.DMA,
        pltpu.SemaphoreType.DMA,
        pltpu.SemaphoreType.DMA,
        pltpu.SemaphoreType.DMA,
        pltpu.SemaphoreType.DMA,
        pltpu.SemaphoreType.DMA,
        pltpu.SemaphoreType.DMA,
    ],
)
def _sc_block(m0_hbm, m1_hbm, se_hbm, do_hbm, cnt_hbm,
              agg0_hbm, agg1_hbm,
              se_b, dst_b, rbA, rbB, zbuf, cnt_v, aggs,
              sem_i, sem_g0, sem_g1, sem_g2, sem_g3,
              sem_s0, sem_s1, sem_s2, sem_s3):
    c = lax.axis_index("c")
    s = lax.axis_index("s")
    zv = jnp.zeros((16,), jnp.float32)

    def zrow(j, carry):
        for k in range(D_HALF // 16):
            zbuf[j, pl.ds(16 * k, 16)] = zv
        return carry

    lax.fori_loop(0, 125, zrow, 0)

    def zcopy(k, carry):
        pltpu.sync_copy(zbuf,
                        aggs.at[pl.ds(s * ROWS_PER_TILE + 125 * k, 125), :])
        return carry

    lax.fori_loop(0, ROWS_PER_TILE // 125, zcopy, 0)
    plsc.subcore_barrier()

    def process(m_ref, agg_ref):
        rbs = (rbA, rbB, rbA, rbB)
        gsems = (sem_g0, sem_g1, sem_g2, sem_g3)
        ssems = (sem_s0, sem_s1, sem_s2, sem_s3)

        def mult(rb, slot):
            sv = jnp.full((16,), slot, jnp.int32)

            def mrow(r4, carry2):
                for u in range(4):
                    r = r4 * 4 + u
                    rv = jnp.full((16,), r + BROW, jnp.int32)
                    ev = plsc.bitcast(plsc.load_gather(se_b, [sv, rv]),
                                      jnp.float32)
                    for k in range(D_HALF // 16):
                        sl = pl.ds(16 * k, 16)
                        rb[r, sl] = rb[r, sl] * ev
                return carry2

            lax.fori_loop(0, BROW // 4, mrow, 0)

        def drain_scatter(k):
            pltpu.make_async_copy(m_ref.at[pl.ds(0, BROW)], rbs[k],
                                  ssems[k]).wait()

        GS = 4
        GR = BROW // GS

        def fire_gather(k):
            for p in range(GS):
                sl = pl.ds(GR * p, GR)
                pltpu.async_copy(m_ref.at[se_b.at[k, sl]],
                                 rbs[k].at[sl, :], gsems[k])

        def wait_gather(k):
            for p in range(GS):
                sl = pl.ds(GR * p, GR)
                pltpu.make_async_copy(m_ref.at[sl], rbs[k].at[sl, :],
                                      gsems[k]).wait()

        for chunk_off in (0, NSUB):
            chunk = s + chunk_off
            pltpu.sync_copy(cnt_hbm.at[chunk], cnt_v)
            cnt = lax.reduce_max(cnt_v[...], axes=(0,))
            nb = (cnt + BROW - 1) // BROW
            nq = (nb + 3) // 4

            def quad(q, carry):
                jbase = 4 * q

                for k in (2, 3):
                    @pl.when(q > 0)
                    def _(k=k):
                        drain_scatter(k)

                for k in range(4):
                    @pl.when(jbase + k < nb)
                    def _(k=k):
                        pltpu.async_copy(se_hbm.at[chunk, jbase + k],
                                         se_b.at[k], sem_i)
                        pltpu.async_copy(do_hbm.at[chunk, jbase + k],
                                         dst_b.at[k], sem_i)
                for k in range(4):
                    @pl.when(jbase + k < nb)
                    def _(k=k):
                        pltpu.make_async_copy(se_hbm.at[chunk, jbase + k],
                                              se_b.at[k], sem_i).wait()
                        pltpu.make_async_copy(do_hbm.at[chunk, jbase + k],
                                              dst_b.at[k], sem_i).wait()

                for k in range(2):
                    @pl.when(jbase + k < nb)
                    def _(k=k):
                        fire_gather(k)

                for k in range(4):
                    @pl.when(jbase + k < nb)
                    def _(k=k):
                        wait_gather(k)
                        mult(rbs[k], k)
                        pltpu.async_copy(rbs[k], aggs.at[dst_b.at[k]],
                                         ssems[k], add=True)
                        if k < 2:
                            @pl.when(jbase + k + 2 < nb)
                            def _(k=k):
                                drain_scatter(k)
                                fire_gather(k + 2)

                return carry

            lax.fori_loop(0, nq, quad, 0)
            rem = nb - 4 * (nq - 1)
            conds = [
                (rem >= 1) & (rem <= 2),
                (rem >= 2) & (rem <= 3),
                rem >= 3,
                rem == 4,
            ]
            for k in range(4):
                @pl.when((nb > 0) & conds[k])
                def _(k=k):
                    drain_scatter(k)
        plsc.subcore_barrier()
        start = pl.multiple_of(s * 624, 8)

        @pl.when(s < NSUB - 1)
        def _():
            sl = pl.ds(start, 624)
            pltpu.sync_copy(aggs.at[sl, :], agg_ref.at[sl, :])

        @pl.when(s == NSUB - 1)
        def _():
            sl = pl.ds((NSUB - 1) * 624, 640)
            pltpu.sync_copy(aggs.at[sl, :], agg_ref.at[sl, :])

    @pl.when(c == 0)
    def _():
        process(m0_hbm, agg0_hbm)

    @pl.when(c == 1)
    def _():
        process(m1_hbm, agg1_hbm)


def kernel(x, pos, edge_index, batch, W_in, W_msg, W_upd, W_mlp1, W_mlp2, W_out):
    srcR = edge_index[0].reshape(NCHUNK, NBIN, BIN)
    dstR = edge_index[1].reshape(NCHUNK, NBIN, BIN)
    pos_flat = pos.reshape(-1)
    se, do_, cnt = _sc_env(pos_flat, srcR, dstR)
    h, m0, m1 = _tc_embed(x, W_in, W_msg[0])
    out = None
    for i in range(4):
        agg0, agg1 = _sc_block(m0, m1, se, do_, cnt)
        wu_t = W_upd[i, :D_HALF]
        wu_b = W_upd[i, D_HALF:]
        if i < 3:
            h, m0, m1 = _tc_update(h, agg0, agg1, wu_t, wu_b, W_msg[i + 1])
        else:
            out = _tc_final(h, agg0, agg1, wu_t, wu_b,
                            batch.reshape(RG, 1, RB), W_mlp1, W_mlp2, W_out)
    return out


# speedup vs baseline: 1.0134x; 1.0134x over previous
import functools
import math

import jax
import jax.numpy as jnp
from jax import lax
from jax.experimental import pallas as pl
from jax.experimental.pallas import tpu as pltpu
from jax.experimental.pallas import tpu_sc as plsc

N = 10000
E = 320000
D_IN = 128
D_EMB = 256
D_HALF = 128
N_GR = 100
N_GR_PAD = 104
CUTOFF = 6.0
PI2 = math.pi * math.pi

NCHUNK = 32
CHUNK = E // NCHUNK
BIN = 80
NBIN = CHUNK // BIN
BROW = 128
NBC = -(-CHUNK // BROW)
RG = 10
RB = N // RG
NSUB = 16
ROWS_PER_TILE = N // NSUB

_COS_SQRT_COEFS = tuple((-1.0) ** k / math.factorial(2 * k) for k in range(12))


def _cos_sqrt(y):
    acc = jnp.full_like(y, _COS_SQRT_COEFS[-1])
    for c in _COS_SQRT_COEFS[-2::-1]:
        acc = acc * y + c
    return acc


def _dot(a, b):
    return jnp.dot(a, b, preferred_element_type=jnp.float32)


def _tc_embed_body(x_ref, wi_ref, wm_ref, h_ref, m0_ref, m1_ref):
    hb = jax.nn.gelu(_dot(x_ref[...], wi_ref[...]))
    h_ref[...] = hb
    mb = jax.nn.gelu(_dot(hb, wm_ref[...]))
    m0_ref[...] = mb[:, :D_HALF]
    m1_ref[...] = mb[:, D_HALF:]


def _tc_embed(x, W_in, Wm0):
    return pl.pallas_call(
        _tc_embed_body,
        grid=(RG,),
        in_specs=[
            pl.BlockSpec((RB, D_IN), lambda i: (i, 0)),
            pl.BlockSpec((D_IN, D_EMB), lambda i: (0, 0)),
            pl.BlockSpec((D_EMB, D_EMB), lambda i: (0, 0)),
        ],
        out_specs=[
            pl.BlockSpec((RB, D_EMB), lambda i: (i, 0)),
            pl.BlockSpec((RB, D_HALF), lambda i: (i, 0)),
            pl.BlockSpec((RB, D_HALF), lambda i: (i, 0)),
        ],
        out_shape=[
            jax.ShapeDtypeStruct((N, D_EMB), jnp.float32),
            jax.ShapeDtypeStruct((N, D_HALF), jnp.float32),
            jax.ShapeDtypeStruct((N, D_HALF), jnp.float32),
        ],
    )(x, W_in, Wm0)


def _tc_update_body(h_ref, a0_ref, a1_ref, wut_ref, wub_ref, wm_ref,
                    hn_ref, m0_ref, m1_ref):
    a = _dot(a0_ref[...], wut_ref[...]) + _dot(a1_ref[...], wub_ref[...])
    hn = h_ref[...] + jax.nn.gelu(a)
    hn_ref[...] = hn
    mb = jax.nn.gelu(_dot(hn, wm_ref[...]))
    m0_ref[...] = mb[:, :D_HALF]
    m1_ref[...] = mb[:, D_HALF:]


def _tc_update(h, agg0, agg1, Wu_t, Wu_b, Wm):
    return pl.pallas_call(
        _tc_update_body,
        grid=(RG,),
        in_specs=[
            pl.BlockSpec((RB, D_EMB), lambda i: (i, 0)),
            pl.BlockSpec((RB, D_HALF), lambda i: (i, 0)),
            pl.BlockSpec((RB, D_HALF), lambda i: (i, 0)),
            pl.BlockSpec((D_HALF, D_EMB), lambda i: (0, 0)),
            pl.BlockSpec((D_HALF, D_EMB), lambda i: (0, 0)),
            pl.BlockSpec((D_EMB, D_EMB), lambda i: (0, 0)),
        ],
        out_specs=[
            pl.BlockSpec((RB, D_EMB), lambda i: (i, 0)),
            pl.BlockSpec((RB, D_HALF), lambda i: (i, 0)),
            pl.BlockSpec((RB, D_HALF), lambda i: (i, 0)),
        ],
        out_shape=[
            jax.ShapeDtypeStruct((N, D_EMB), jnp.float32),
            jax.ShapeDtypeStruct((N, D_HALF), jnp.float32),
            jax.ShapeDtypeStruct((N, D_HALF), jnp.float32),
        ],
    )(h, agg0, agg1, Wu_t, Wu_b, Wm)


def _tc_final_body(h_ref, a0_ref, a1_ref, wut_ref, wub_ref, b_ref,
                   w1_ref, w2_ref, w3_ref, out_ref, g_acc):
    i = pl.program_id(0)
    a = _dot(a0_ref[...], wut_ref[...]) + _dot(a1_ref[...], wub_ref[...])
    hn = h_ref[...] + jax.nn.gelu(a)
    seg = b_ref[0]
    onehot = (lax.broadcasted_iota(jnp.int32, (N_GR_PAD, RB), 0) == seg
              ).astype(jnp.float32)

    @pl.when(i == 0)
    def _():
        g_acc[...] = jnp.zeros_like(g_acc)

    g_acc[...] += _dot(onehot, hn)

    @pl.when(i == RG - 1)
    def _():
        g = jax.nn.gelu(_dot(g_acc[...], w1_ref[...]))
        g = jax.nn.gelu(_dot(g, w2_ref[...]))
        out_ref[...] = _dot(g, w3_ref[...])[:N_GR]


def _tc_final(h, agg0, agg1, Wu_t, Wu_b, batch3, W1, W2, W3):
    return pl.pallas_call(
        _tc_final_body,
        grid=(RG,),
        in_specs=[
            pl.BlockSpec((RB, D_EMB), lambda i: (i, 0)),
            pl.BlockSpec((RB, D_HALF), lambda i: (i, 0)),
            pl.BlockSpec((RB, D_HALF), lambda i: (i, 0)),
            pl.BlockSpec((D_HALF, D_EMB), lambda i: (0, 0)),
            pl.BlockSpec((D_HALF, D_EMB), lambda i: (0, 0)),
            pl.BlockSpec((1, 1, RB), lambda i: (i, 0, 0)),
            pl.BlockSpec((D_EMB, D_EMB), lambda i: (0, 0)),
            pl.BlockSpec((D_EMB, D_EMB), lambda i: (0, 0)),
            pl.BlockSpec((D_EMB, D_IN), lambda i: (0, 0)),
        ],
        out_specs=pl.BlockSpec((N_GR, D_IN), lambda i: (0, 0)),
        out_shape=jax.ShapeDtypeStruct((N_GR, D_IN), jnp.float32),
        scratch_shapes=[pltpu.VMEM((N_GR_PAD, D_EMB), jnp.float32)],
    )(h, agg0, agg1, Wu_t, Wu_b, batch3, W1, W2, W3)


_MESH = plsc.VectorSubcoreMesh(core_axis_name="c", subcore_axis_name="s")
_SC_PARAMS = pltpu.CompilerParams(needs_layout_passes=False)


@functools.partial(
    pl.kernel,
    out_type=[
        jax.ShapeDtypeStruct((NCHUNK, NBC, 2 * BROW), jnp.int32),
        jax.ShapeDtypeStruct((NCHUNK, NBC, BROW), jnp.int32),
        jax.ShapeDtypeStruct((NCHUNK, 16), jnp.int32),
    ],
    mesh=_MESH,
    compiler_params=_SC_PARAMS,
    scratch_types=[
        pltpu.VMEM((3 * N,), jnp.float32),
        pltpu.VMEM((NBIN, BIN), jnp.int32),
        pltpu.VMEM((NBIN, BIN), jnp.int32),
        pltpu.VMEM((NBC, 2 * BROW), jnp.int32),
        pltpu.VMEM((NBC, BROW), jnp.int32),
        pltpu.VMEM((16,), jnp.int32),
    ],
)
def _sc_env(pos_hbm, srcR_hbm, dstR_hbm, se_hbm, do_hbm, cnt_hbm,
            pos_v, src_v, dst_v, se_v, do_v, cnt_v):
    c = lax.axis_index("c")
    s = lax.axis_index("s")
    wid = s * 2 + c
    pltpu.sync_copy(pos_hbm, pos_v)
    pltpu.sync_copy(srcR_hbm.at[wid], src_v)
    pltpu.sync_copy(dstR_hbm.at[wid], dst_v)
    iz = jnp.zeros((16,), jnp.int32)
    i1 = jnp.full((16,), 1, jnp.int32)
    i2 = jnp.full((16,), 2, jnp.int32)
    scale = PI2 / (CUTOFF * CUTOFF)

    def zrow(j, carry):
        for k in range(2 * BROW // 16):
            se_v[j, pl.ds(16 * k, 16)] = iz
        for k in range(BROW // 16):
            do_v[j, pl.ds(16 * k, 16)] = iz
        return carry

    lax.fori_loop(0, NBC, zrow, 0)

    def row(j, off):
        for k in range(BIN // 16):
            sl = pl.ds(16 * k, 16)
            sv = src_v[j, sl]
            tv = dst_v[j, sl]
            s3 = sv * 3
            t3 = tv * 3
            dx = (plsc.load_gather(pos_v, [s3 + iz])
                  - plsc.load_gather(pos_v, [t3 + iz]))
            dy = (plsc.load_gather(pos_v, [s3 + i1])
                  - plsc.load_gather(pos_v, [t3 + i1]))
            dz = (plsc.load_gather(pos_v, [s3 + i2])
                  - plsc.load_gather(pos_v, [t3 + i2]))
            s2 = dx * dx + dy * dy + dz * dz + 1e-8
            y = s2 * scale
            ev = 0.5 * (_cos_sqrt(y) + 1.0)
            ev = jnp.maximum(ev, 0.0)
            keep = y < PI2
            kin = jnp.where(keep, 1, 0)
            pos = off + plsc.cumsum(kin) - 1
            jb = pos // BROW
            lane = pos - jb * BROW
            plsc.store_scatter(se_v, [jb, lane], sv, mask=keep)
            plsc.store_scatter(se_v, [jb, lane + BROW],
                               plsc.bitcast(ev, jnp.int32), mask=keep)
            plsc.store_scatter(do_v, [jb, lane], tv, mask=keep)
            off = off + plsc.all_reduce_population_count(keep)
        return off

    off = lax.fori_loop(0, NBIN, row, iz)
    cnt_v[...] = off
    pltpu.sync_copy(se_v, se_hbm.at[wid])
    pltpu.sync_copy(do_v, do_hbm.at[wid])
    pltpu.sync_copy(cnt_v, cnt_hbm.at[wid])


@functools.partial(
    pl.kernel,
    out_type=[
        jax.ShapeDtypeStruct((N, D_HALF), jnp.float32),
        jax.ShapeDtypeStruct((N, D_HALF), jnp.float32),
    ],
    mesh=_MESH,
    compiler_params=_SC_PARAMS,
    scratch_types=[
        pltpu.VMEM((4, 2 * BROW), jnp.int32),
        pltpu.VMEM((4, BROW), jnp.int32),
        pltpu.VMEM((BROW, D_HALF), jnp.float32),
        pltpu.VMEM((BROW, D_HALF), jnp.float32),
        pltpu.VMEM((BROW, D_HALF), jnp.float32),
        pltpu.VMEM((16,), jnp.int32),
        pltpu.VMEM_SHARED((N, D_HALF), jnp.float32),
        pltpu.SemaphoreType.DMA,
        pltpu.SemaphoreType.DMA,
        pltpu.SemaphoreType.DMA,
        pltpu.SemaphoreType.DMA,
        pltpu.SemaphoreType.DMA,
        pltpu.SemaphoreType.DMA,
        pltpu.SemaphoreType.DMA,
        pltpu.SemaphoreType.DMA,
        pltpu.SemaphoreType.DMA,
    ],
)
def _sc_block(m0_hbm, m1_hbm, se_hbm, do_hbm, cnt_hbm,
              agg0_hbm, agg1_hbm,
              se_b, dst_b, rbA, rbB, rbC, cnt_v, aggs,
              sem_i, sem_g0, sem_g1, sem_g2, sem_g3,
              sem_s0, sem_s1, sem_s2, sem_s3):
    c = lax.axis_index("c")
    s = lax.axis_index("s")
    zv = jnp.zeros((16,), jnp.float32)

    def zrow(j, carry):
        for k in range(D_HALF // 16):
            rbA[j, pl.ds(16 * k, 16)] = zv
        return carry

    lax.fori_loop(0, BROW, zrow, 0)

    def zcopy(k, carry):
        pltpu.sync_copy(rbA.at[pl.ds(0, 125), :],
                        aggs.at[pl.ds(s * ROWS_PER_TILE + 125 * k, 125), :])
        return carry

    lax.fori_loop(0, ROWS_PER_TILE // 125, zcopy, 0)
    plsc.subcore_barrier()

    def process(m_ref, agg_ref):
        rbs = (rbA, rbB, rbC, rbA)
        gsems = (sem_g0, sem_g1, sem_g2, sem_g3)
        ssems = (sem_s0, sem_s1, sem_s2, sem_s3)

        def mult(rb, slot):
            sv = jnp.full((16,), slot, jnp.int32)

            def mrow(r4, carry2):
                for u in range(4):
                    r = r4 * 4 + u
                    rv = jnp.full((16,), r + BROW, jnp.int32)
                    ev = plsc.bitcast(plsc.load_gather(se_b, [sv, rv]),
                                      jnp.float32)
                    for k in range(D_HALF // 16):
                        sl = pl.ds(16 * k, 16)
                        rb[r, sl] = rb[r, sl] * ev
                return carry2

            lax.fori_loop(0, BROW // 4, mrow, 0)

        def drain_scatter(k):
            pltpu.make_async_copy(m_ref.at[pl.ds(0, BROW)], rbs[k],
                                  ssems[k]).wait()

        GS = 4
        GR = BROW // GS

        def fire_gather(k):
            for p in range(GS):
                sl = pl.ds(GR * p, GR)
                pltpu.async_copy(m_ref.at[se_b.at[k, sl]],
                                 rbs[k].at[sl, :], gsems[k])

        def wait_gather(k):
            for p in range(GS):
                sl = pl.ds(GR * p, GR)
                pltpu.make_async_copy(m_ref.at[sl], rbs[k].at[sl, :],
                                      gsems[k]).wait()

        for chunk_off in (0, NSUB):
            chunk = s + chunk_off
            pltpu.sync_copy(cnt_hbm.at[chunk], cnt_v)
            cnt = lax.reduce_max(cnt_v[...], axes=(0,))
            nb = (cnt + BROW - 1) // BROW
            nq = (nb + 3) // 4

            def quad(q, carry):
                jbase = 4 * q

                for k in (1, 2, 3):
                    @pl.when(q > 0)
                    def _(k=k):
                        drain_scatter(k)

                for k in range(4):
                    @pl.when(jbase + k < nb)
                    def _(k=k):
                        pltpu.async_copy(se_hbm.at[chunk, jbase + k],
                                         se_b.at[k], sem_i)
                        pltpu.async_copy(do_hbm.at[chunk, jbase + k],
                                         dst_b.at[k], sem_i)
                for k in range(4):
                    @pl.when(jbase + k < nb)
                    def _(k=k):
                        pltpu.make_async_copy(se_hbm.at[chunk, jbase + k],
                                              se_b.at[k], sem_i).wait()
                        pltpu.make_async_copy(do_hbm.at[chunk, jbase + k],
                                              dst_b.at[k], sem_i).wait()

                for k in range(3):
                    @pl.when(jbase + k < nb)
                    def _(k=k):
                        fire_gather(k)

                for k in range(4):
                    @pl.when(jbase + k < nb)
                    def _(k=k):
                        wait_gather(k)
                        mult(rbs[k], k)
                        pltpu.async_copy(rbs[k], aggs.at[dst_b.at[k]],
                                         ssems[k], add=True)
                        if k == 0:
                            @pl.when(jbase + 3 < nb)
                            def _():
                                drain_scatter(0)
                                fire_gather(3)

                return carry

            lax.fori_loop(0, nq, quad, 0)
            rem = nb - 4 * (nq - 1)
            conds = [
                (rem >= 1) & (rem <= 3),
                rem >= 2,
                rem >= 3,
                rem == 4,
            ]
            for k in range(4):
                @pl.when((nb > 0) & conds[k])
                def _(k=k):
                    drain_scatter(k)
        plsc.subcore_barrier()
        start = pl.multiple_of(s * 624, 8)

        @pl.when(s < NSUB - 1)
        def _():
            sl = pl.ds(start, 624)
            pltpu.sync_copy(aggs.at[sl, :], agg_ref.at[sl, :])

        @pl.when(s == NSUB - 1)
        def _():
            sl = pl.ds((NSUB - 1) * 624, 640)
            pltpu.sync_copy(aggs.at[sl, :], agg_ref.at[sl, :])

    @pl.when(c == 0)
    def _():
        process(m0_hbm, agg0_hbm)

    @pl.when(c == 1)
    def _():
        process(m1_hbm, agg1_hbm)


def kernel(x, pos, edge_index, batch, W_in, W_msg, W_upd, W_mlp1, W_mlp2, W_out):
    srcR = edge_index[0].reshape(NCHUNK, NBIN, BIN)
    dstR = edge_index[1].reshape(NCHUNK, NBIN, BIN)
    pos_flat = pos.reshape(-1)
    se, do_, cnt = _sc_env(pos_flat, srcR, dstR)
    h, m0, m1 = _tc_embed(x, W_in, W_msg[0])
    out = None
    for i in range(4):
        agg0, agg1 = _sc_block(m0, m1, se, do_, cnt)
        wu_t = W_upd[i, :D_HALF]
        wu_b = W_upd[i, D_HALF:]
        if i < 3:
            h, m0, m1 = _tc_update(h, agg0, agg1, wu_t, wu_b, W_msg[i + 1])
        else:
            out = _tc_final(h, agg0, agg1, wu_t, wu_b,
                            batch.reshape(RG, 1, RB), W_mlp1, W_mlp2, W_out)
    return out
